# baseline (speedup 1.0000x reference)
import functools

import jax
import jax.numpy as jnp
from jax import lax
from jax.experimental import pallas as pl
from jax.experimental.pallas import tpu as pltpu
from jax.experimental.pallas import tpu_sc as plsc

_NTAP = 4
_NW = 32
_KC = 80
_EB = 2000
_EB2 = 3200


def _sc_gather(table, idx):
    n, = idx.shape
    _, c = table.shape
    per_w = n // _NW
    nchunk = per_w // _KC

    mesh = plsc.VectorSubcoreMesh(core_axis_name="c", subcore_axis_name="s")

    @functools.partial(
        pl.kernel,
        mesh=mesh,
        compiler_params=pltpu.CompilerParams(use_tc_tiling_on_sc=False),
        out_type=jax.ShapeDtypeStruct((n, c), table.dtype),
        scratch_types=[
            pltpu.VMEM((2, _KC), jnp.int32),
            pltpu.VMEM((2, _KC, c), table.dtype),
            pltpu.SemaphoreType.DMA,
            pltpu.SemaphoreType.DMA,
            pltpu.SemaphoreType.DMA,
        ],
    )
    def gather_kernel(table_hbm, idx_hbm, out_hbm, idx_v, rows_v,
                      sem_i, sem_g, sem_w):
        wid = lax.axis_index("s") * 2 + lax.axis_index("c")
        base_w = wid * per_w

        pltpu.async_copy(idx_hbm.at[pl.ds(base_w, _KC)], idx_v.at[0], sem_i)

        def chunk_step(t, carry):
            s = t % 2
            base = base_w + t * _KC
            pltpu.make_async_copy(
                idx_hbm.at[pl.ds(base, _KC)], idx_v.at[s], sem_i).wait()

            @pl.when(t + 1 < nchunk)
            def _prefetch():
                pltpu.async_copy(
                    idx_hbm.at[pl.ds(base + _KC, _KC)], idx_v.at[1 - s], sem_i)

            @pl.when(t >= 2)
            def _reclaim():
                pltpu.make_async_copy(
                    rows_v.at[s], out_hbm.at[pl.ds(base - 2 * _KC, _KC)],
                    sem_w).wait()

            pltpu.async_copy(table_hbm.at[idx_v.at[s]], rows_v.at[s],
                             sem_g).wait()
            pltpu.async_copy(rows_v.at[s], out_hbm.at[pl.ds(base, _KC)], sem_w)
            return carry

        lax.fori_loop(0, nchunk, chunk_step, 0)
        pltpu.make_async_copy(
            rows_v.at[0], out_hbm.at[pl.ds(base_w, _KC)], sem_w).wait()
        pltpu.make_async_copy(
            rows_v.at[0], out_hbm.at[pl.ds(base_w, _KC)], sem_w).wait()

    return gather_kernel(table, idx)


def _tc_conv_stats(xt, taps, wc):
    e, c = xt.shape

    def body(xt_ref, taps_ref, wc_ref, y_ref, s1_ref, s2_ref):
        i = pl.program_id(0)
        x = xt_ref[...]
        a = taps_ref[0]
        bb = taps_ref[1]
        cc = taps_ref[2]
        dd = taps_ref[3]
        h = jnp.concatenate(
            [x, a + cc, bb + dd, jnp.abs(a - cc), jnp.abs(bb - dd)], axis=1)
        y = jnp.dot(h, wc_ref[...], preferred_element_type=jnp.float32)
        y_ref[...] = y

        @pl.when(i == 0)
        def _init():
            s1_ref[...] = jnp.zeros_like(s1_ref)
            s2_ref[...] = jnp.zeros_like(s2_ref)

        s1_ref[...] += jnp.sum(y, axis=0, keepdims=True)
        s2_ref[...] += jnp.sum(y * y, axis=0, keepdims=True)

    return pl.pallas_call(
        body,
        grid=(e // _EB,),
        in_specs=[
            pl.BlockSpec((_EB, c), lambda i: (i, 0)),
            pl.BlockSpec((_NTAP, _EB, c), lambda i: (0, i, 0)),
            pl.BlockSpec((5 * c, c), lambda i: (0, 0)),
        ],
        out_specs=[
            pl.BlockSpec((_EB, c), lambda i: (i, 0)),
            pl.BlockSpec((1, c), lambda i: (0, 0)),
            pl.BlockSpec((1, c), lambda i: (0, 0)),
        ],
        out_shape=[
            jax.ShapeDtypeStruct((e, c), jnp.float32),
            jax.ShapeDtypeStruct((1, c), jnp.float32),
            jax.ShapeDtypeStruct((1, c), jnp.float32),
        ],
    )(xt, taps, wc)


def _tc_bn_relu(y, scale, shift):
    e, c = y.shape

    def body(y_ref, sc_ref, sh_ref, o_ref):
        z = jnp.maximum(y_ref[...] * sc_ref[...] + sh_ref[...], 0.0)
        o_ref[...] = z.T

    return pl.pallas_call(
        body,
        grid=(e // _EB2,),
        in_specs=[
            pl.BlockSpec((_EB2, c), lambda i: (i, 0)),
            pl.BlockSpec((1, c), lambda i: (0, 0)),
            pl.BlockSpec((1, c), lambda i: (0, 0)),
        ],
        out_specs=pl.BlockSpec((c, _EB2), lambda i: (0, i)),
        out_shape=jax.ShapeDtypeStruct((c, e), jnp.float32),
    )(y, scale, shift)


def kernel(x, gemm, W, b, gamma, beta):
    _, c_in, e = x.shape
    c_out = W.shape[0]

    xt = jnp.swapaxes(x[0], 0, 1).astype(jnp.bfloat16)
    idx = jnp.swapaxes(gemm[0], 0, 1).reshape(-1)
    xt_words = lax.bitcast_convert_type(
        xt.reshape(e, c_in // 2, 2), jnp.int32)
    taps_words = _sc_gather(xt_words, idx)
    taps = lax.bitcast_convert_type(
        taps_words, jnp.bfloat16).reshape(_NTAP, e, c_in)

    wc = jnp.transpose(W, (2, 1, 0)).reshape(5 * c_in, c_out).astype(jnp.bfloat16)
    y, s1, s2 = _tc_conv_stats(xt, taps, wc)

    mean = s1[0] / e
    var = s2[0] / e - mean * mean
    inv = gamma / jnp.sqrt(var + 1e-5)
    scale = inv[None]
    shift = (beta - mean * inv)[None]

    out = _tc_bn_relu(y, scale, shift)[None]
    return (out, gemm)


# speedup vs baseline: 4.1257x; 4.1257x over previous
import functools

import jax
import jax.numpy as jnp
from jax import lax
from jax.experimental import pallas as pl
from jax.experimental.pallas import tpu as pltpu
from jax.experimental.pallas import tpu_sc as plsc

_NTAP = 4
_NW = 32
_KC = 80
_EB = 2000
_EB2 = 3200


def _sc_gather(table, idx):
    n, = idx.shape
    _, c = table.shape
    per_w = n // _NW
    nchunk = per_w // _KC

    mesh = plsc.VectorSubcoreMesh(core_axis_name="c", subcore_axis_name="s")

    @functools.partial(
        pl.kernel,
        mesh=mesh,
        out_type=jax.ShapeDtypeStruct((n, c), table.dtype),
        scratch_types=[
            pltpu.VMEM((2, _KC), jnp.int32),
            pltpu.VMEM((2, _KC, c), table.dtype),
            pltpu.SemaphoreType.DMA,
            pltpu.SemaphoreType.DMA,
            pltpu.SemaphoreType.DMA,
        ],
    )
    def gather_kernel(table_hbm, idx_hbm, out_hbm, idx_v, rows_v,
                      sem_i, sem_g, sem_w):
        wid = lax.axis_index("s") * 2 + lax.axis_index("c")
        base_w = wid * per_w

        pltpu.async_copy(idx_hbm.at[pl.ds(base_w, _KC)], idx_v.at[0], sem_i)

        def chunk_step(t, carry):
            s = t % 2
            base = base_w + t * _KC
            pltpu.make_async_copy(
                idx_hbm.at[pl.ds(base, _KC)], idx_v.at[s], sem_i).wait()

            @pl.when(t + 1 < nchunk)
            def _prefetch():
                pltpu.async_copy(
                    idx_hbm.at[pl.ds(base + _KC, _KC)], idx_v.at[1 - s], sem_i)

            @pl.when(t >= 2)
            def _reclaim():
                pltpu.make_async_copy(
                    rows_v.at[s], out_hbm.at[pl.ds(base - 2 * _KC, _KC)],
                    sem_w).wait()

            pltpu.async_copy(table_hbm.at[idx_v.at[s]], rows_v.at[s],
                             sem_g).wait()
            pltpu.async_copy(rows_v.at[s], out_hbm.at[pl.ds(base, _KC)], sem_w)
            return carry

        lax.fori_loop(0, nchunk, chunk_step, 0)
        pltpu.make_async_copy(
            rows_v.at[0], out_hbm.at[pl.ds(base_w, _KC)], sem_w).wait()
        pltpu.make_async_copy(
            rows_v.at[0], out_hbm.at[pl.ds(base_w, _KC)], sem_w).wait()

    return gather_kernel(table, idx)


def _tc_conv_stats(xt, taps, wc):
    e, c = xt.shape

    def body(xt_ref, taps_ref, wc_ref, y_ref, s1_ref, s2_ref):
        i = pl.program_id(0)
        bf = jnp.bfloat16
        x = xt_ref[...].astype(bf)
        a = taps_ref[0]
        bb = taps_ref[1]
        cc = taps_ref[2]
        dd = taps_ref[3]
        h = jnp.concatenate(
            [x, (a + cc).astype(bf), (bb + dd).astype(bf),
             jnp.abs(a - cc).astype(bf), jnp.abs(bb - dd).astype(bf)], axis=1)
        y = jnp.dot(h, wc_ref[...], preferred_element_type=jnp.float32)
        y_ref[...] = y

        @pl.when(i == 0)
        def _init():
            s1_ref[...] = jnp.zeros_like(s1_ref)
            s2_ref[...] = jnp.zeros_like(s2_ref)

        s1_ref[...] += jnp.sum(y, axis=0, keepdims=True)
        s2_ref[...] += jnp.sum(y * y, axis=0, keepdims=True)

    return pl.pallas_call(
        body,
        grid=(e // _EB,),
        in_specs=[
            pl.BlockSpec((_EB, c), lambda i: (i, 0)),
            pl.BlockSpec((_NTAP, _EB, c), lambda i: (0, i, 0)),
            pl.BlockSpec((5 * c, c), lambda i: (0, 0)),
        ],
        out_specs=[
            pl.BlockSpec((_EB, c), lambda i: (i, 0)),
            pl.BlockSpec((1, c), lambda i: (0, 0)),
            pl.BlockSpec((1, c), lambda i: (0, 0)),
        ],
        out_shape=[
            jax.ShapeDtypeStruct((e, c), jnp.float32),
            jax.ShapeDtypeStruct((1, c), jnp.float32),
            jax.ShapeDtypeStruct((1, c), jnp.float32),
        ],
    )(xt, taps, wc)


def _tc_bn_relu(y, scale, shift):
    e, c = y.shape

    def body(y_ref, sc_ref, sh_ref, o_ref):
        z = jnp.maximum(y_ref[...] * sc_ref[...] + sh_ref[...], 0.0)
        o_ref[...] = z.T

    return pl.pallas_call(
        body,
        grid=(e // _EB2,),
        in_specs=[
            pl.BlockSpec((_EB2, c), lambda i: (i, 0)),
            pl.BlockSpec((1, c), lambda i: (0, 0)),
            pl.BlockSpec((1, c), lambda i: (0, 0)),
        ],
        out_specs=pl.BlockSpec((c, _EB2), lambda i: (0, i)),
        out_shape=jax.ShapeDtypeStruct((c, e), jnp.float32),
    )(y, scale, shift)


def kernel(x, gemm, W, b, gamma, beta):
    _, c_in, e = x.shape
    c_out = W.shape[0]

    xt = jnp.swapaxes(x[0], 0, 1)
    idx = jnp.swapaxes(gemm[0], 0, 1).reshape(-1)
    taps = _sc_gather(xt, idx).reshape(_NTAP, e, c_in)

    wc = jnp.transpose(W, (2, 1, 0)).reshape(5 * c_in, c_out).astype(jnp.bfloat16)
    y, s1, s2 = _tc_conv_stats(xt, taps, wc)

    mean = s1[0] / e
    var = s2[0] / e - mean * mean
    inv = gamma / jnp.sqrt(var + 1e-5)
    scale = inv[None]
    shift = (beta - mean * inv)[None]

    out = _tc_bn_relu(y, scale, shift)[None]
    return (out, gemm)


# speedup vs baseline: 5.0132x; 1.2151x over previous
import functools

import jax
import jax.numpy as jnp
from jax import lax
from jax.experimental import pallas as pl
from jax.experimental.pallas import tpu as pltpu
from jax.experimental.pallas import tpu_sc as plsc

_NTAP = 4
_NW = 32
_KC = 128
_EB = 2000
_EB2 = 3200


def _sc_gather(table, idx):
    n, = idx.shape
    _, c = table.shape
    per_w = n // _NW
    nfull = per_w // _KC
    tail = per_w - nfull * _KC
    assert nfull % 3 == 0 and nfull >= 6

    mesh = plsc.VectorSubcoreMesh(core_axis_name="c", subcore_axis_name="s")

    @functools.partial(
        pl.kernel,
        mesh=mesh,
        out_type=jax.ShapeDtypeStruct((n, c), table.dtype),
        scratch_types=[
            pltpu.VMEM((per_w,), jnp.int32),
            pltpu.VMEM((_KC, c), table.dtype),
            pltpu.VMEM((_KC, c), table.dtype),
            pltpu.VMEM((_KC, c), table.dtype),
            pltpu.SemaphoreType.DMA,
            pltpu.SemaphoreType.DMA,
            pltpu.SemaphoreType.DMA,
            pltpu.SemaphoreType.DMA,
            pltpu.SemaphoreType.DMA,
            pltpu.SemaphoreType.DMA,
        ],
    )
    def gather_kernel(table_hbm, idx_hbm, out_hbm, idx_v, r0, r1, r2,
                      g0, g1, g2, w0, w1, w2):
        rows = (r0, r1, r2)
        semg = (g0, g1, g2)
        semw = (w0, w1, w2)
        wid = lax.axis_index("s") * 2 + lax.axis_index("c")
        base_w = wid * per_w

        pltpu.sync_copy(idx_hbm.at[pl.ds(base_w, per_w)], idx_v)

        def g_idx(m):
            return idx_v.at[pl.ds(m * _KC, _KC)]

        pltpu.async_copy(table_hbm.at[g_idx(0)], r0, g0)
        pltpu.async_copy(table_hbm.at[g_idx(1)], r1, g1)

        def step(j, carry):
            for k in range(3):
                m = 3 * j + k
                s2 = (k + 2) % 3

                @pl.when(m + 2 < nfull)
                def _launch():
                    @pl.when(m >= 1)
                    def _reclaim():
                        pltpu.make_async_copy(
                            rows[s2], out_hbm.at[pl.ds(base_w, _KC)],
                            semw[s2]).wait()
                    pltpu.async_copy(
                        table_hbm.at[g_idx(m + 2)], rows[s2], semg[s2])

                pltpu.make_async_copy(
                    table_hbm.at[g_idx(m)], rows[k], semg[k]).wait()
                pltpu.async_copy(
                    rows[k], out_hbm.at[pl.ds(base_w + m * _KC, _KC)], semw[k])
            return carry

        lax.fori_loop(0, nfull // 3, step, 0)
        for m in (nfull - 3, nfull - 2, nfull - 1):
            pltpu.make_async_copy(
                rows[m % 3], out_hbm.at[pl.ds(base_w, _KC)], semw[m % 3]).wait()
        if tail:
            tb = base_w + nfull * _KC
            pltpu.async_copy(
                table_hbm.at[idx_v.at[pl.ds(nfull * _KC, tail)]],
                r0.at[pl.ds(0, tail)], g0).wait()
            pltpu.sync_copy(r0.at[pl.ds(0, tail)], out_hbm.at[pl.ds(tb, tail)])

    return gather_kernel(table, idx)


def _tc_conv_stats(xt, taps, wc):
    e, c = xt.shape

    def body(xt_ref, taps_ref, wc_ref, y_ref, s1_ref, s2_ref):
        i = pl.program_id(0)
        bf = jnp.bfloat16
        x = xt_ref[...].astype(bf)
        a = taps_ref[0]
        bb = taps_ref[1]
        cc = taps_ref[2]
        dd = taps_ref[3]
        h = jnp.concatenate(
            [x, (a + cc).astype(bf), (bb + dd).astype(bf),
             jnp.abs(a - cc).astype(bf), jnp.abs(bb - dd).astype(bf)], axis=1)
        y = jnp.dot(h, wc_ref[...], preferred_element_type=jnp.float32)
        y_ref[...] = y

        @pl.when(i == 0)
        def _init():
            s1_ref[...] = jnp.zeros_like(s1_ref)
            s2_ref[...] = jnp.zeros_like(s2_ref)

        s1_ref[...] += jnp.sum(y, axis=0, keepdims=True)
        s2_ref[...] += jnp.sum(y * y, axis=0, keepdims=True)

    return pl.pallas_call(
        body,
        grid=(e // _EB,),
        in_specs=[
            pl.BlockSpec((_EB, c), lambda i: (i, 0)),
            pl.BlockSpec((_NTAP, _EB, c), lambda i: (0, i, 0)),
            pl.BlockSpec((5 * c, c), lambda i: (0, 0)),
        ],
        out_specs=[
            pl.BlockSpec((_EB, c), lambda i: (i, 0)),
            pl.BlockSpec((1, c), lambda i: (0, 0)),
            pl.BlockSpec((1, c), lambda i: (0, 0)),
        ],
        out_shape=[
            jax.ShapeDtypeStruct((e, c), jnp.float32),
            jax.ShapeDtypeStruct((1, c), jnp.float32),
            jax.ShapeDtypeStruct((1, c), jnp.float32),
        ],
    )(xt, taps, wc)


def _tc_bn_relu(y, scale, shift):
    e, c = y.shape

    def body(y_ref, sc_ref, sh_ref, o_ref):
        z = jnp.maximum(y_ref[...] * sc_ref[...] + sh_ref[...], 0.0)
        o_ref[...] = z.T

    return pl.pallas_call(
        body,
        grid=(e // _EB2,),
        in_specs=[
            pl.BlockSpec((_EB2, c), lambda i: (i, 0)),
            pl.BlockSpec((1, c), lambda i: (0, 0)),
            pl.BlockSpec((1, c), lambda i: (0, 0)),
        ],
        out_specs=pl.BlockSpec((c, _EB2), lambda i: (0, i)),
        out_shape=jax.ShapeDtypeStruct((c, e), jnp.float32),
    )(y, scale, shift)


def kernel(x, gemm, W, b, gamma, beta):
    _, c_in, e = x.shape
    c_out = W.shape[0]

    xt = jnp.swapaxes(x[0], 0, 1)
    idx = jnp.swapaxes(gemm[0], 0, 1).reshape(-1)
    taps = _sc_gather(xt, idx).reshape(_NTAP, e, c_in)

    wc = jnp.transpose(W, (2, 1, 0)).reshape(5 * c_in, c_out).astype(jnp.bfloat16)
    y, s1, s2 = _tc_conv_stats(xt, taps, wc)

    mean = s1[0] / e
    var = s2[0] / e - mean * mean
    inv = gamma / jnp.sqrt(var + 1e-5)
    scale = inv[None]
    shift = (beta - mean * inv)[None]

    out = _tc_bn_relu(y, scale, shift)[None]
    return (out, gemm)


# speedup vs baseline: 5.0802x; 1.0134x over previous
import functools

import jax
import jax.numpy as jnp
from jax import lax
from jax.experimental import pallas as pl
from jax.experimental.pallas import tpu as pltpu
from jax.experimental.pallas import tpu_sc as plsc

_NTAP = 4
_NW = 32
_KC = 128
_EB = 2000
_EB2 = 3200


def _sc_gather(table, idx):
    n, = idx.shape
    _, c = table.shape
    per_w = n // _NW
    nfull = per_w // _KC
    tail = per_w - nfull * _KC
    assert nfull % 3 == 0 and nfull >= 6

    mesh = plsc.VectorSubcoreMesh(core_axis_name="c", subcore_axis_name="s")

    @functools.partial(
        pl.kernel,
        mesh=mesh,
        out_type=jax.ShapeDtypeStruct((n, c), table.dtype),
        scratch_types=[
            pltpu.VMEM((per_w,), jnp.int32),
            pltpu.VMEM((_KC, c), table.dtype),
            pltpu.VMEM((_KC, c), table.dtype),
            pltpu.VMEM((_KC, c), table.dtype),
            pltpu.SemaphoreType.DMA,
            pltpu.SemaphoreType.DMA,
            pltpu.SemaphoreType.DMA,
            pltpu.SemaphoreType.DMA,
            pltpu.SemaphoreType.DMA,
            pltpu.SemaphoreType.DMA,
        ],
    )
    def gather_kernel(table_hbm, idx_hbm, out_hbm, idx_v, r0, r1, r2,
                      g0, g1, g2, w0, w1, w2):
        rows = (r0, r1, r2)
        semg = (g0, g1, g2)
        semw = (w0, w1, w2)
        wid = lax.axis_index("s") * 2 + lax.axis_index("c")
        base_w = wid * per_w

        pltpu.sync_copy(idx_hbm.at[pl.ds(base_w, per_w)], idx_v)

        def g_idx(m):
            return idx_v.at[pl.ds(m * _KC, _KC)]

        pltpu.async_copy(table_hbm.at[g_idx(0)], r0, g0)
        pltpu.async_copy(table_hbm.at[g_idx(1)], r1, g1)

        def step(j, carry):
            for k in range(3):
                m = 3 * j + k
                s2 = (k + 2) % 3

                @pl.when(m + 2 < nfull)
                def _launch():
                    @pl.when(m >= 1)
                    def _reclaim():
                        pltpu.make_async_copy(
                            rows[s2], out_hbm.at[pl.ds(base_w, _KC)],
                            semw[s2]).wait()
                    pltpu.async_copy(
                        table_hbm.at[g_idx(m + 2)], rows[s2], semg[s2])

                pltpu.make_async_copy(
                    table_hbm.at[g_idx(m)], rows[k], semg[k]).wait()
                pltpu.async_copy(
                    rows[k], out_hbm.at[pl.ds(base_w + m * _KC, _KC)], semw[k])
            return carry

        lax.fori_loop(0, nfull // 3, step, 0)
        for m in (nfull - 3, nfull - 2, nfull - 1):
            pltpu.make_async_copy(
                rows[m % 3], out_hbm.at[pl.ds(base_w, _KC)], semw[m % 3]).wait()
        if tail:
            tb = base_w + nfull * _KC
            pltpu.async_copy(
                table_hbm.at[idx_v.at[pl.ds(nfull * _KC, tail)]],
                r0.at[pl.ds(0, tail)], g0).wait()
            pltpu.sync_copy(r0.at[pl.ds(0, tail)], out_hbm.at[pl.ds(tb, tail)])

    return gather_kernel(table, idx)


def _tc_conv_stats(xt, taps_h, wc, y_prev, s1_init, s2_init, half, nhalf):
    e, c = xt.shape
    nbh = (e // nhalf) // _EB
    off = half * nbh

    def body(xt_ref, taps_ref, wc_ref, s1i_ref, s2i_ref, *rest):
        y_ref, s1_ref, s2_ref = rest[-3:]
        i = pl.program_id(0)
        bf = jnp.bfloat16
        x = xt_ref[...].astype(bf)
        a = taps_ref[0]
        bb = taps_ref[1]
        cc = taps_ref[2]
        dd = taps_ref[3]
        h = jnp.concatenate(
            [x, (a + cc).astype(bf), (bb + dd).astype(bf),
             jnp.abs(a - cc).astype(bf), jnp.abs(bb - dd).astype(bf)], axis=1)
        y = jnp.dot(h, wc_ref[...], preferred_element_type=jnp.float32)
        y_ref[...] = y

        @pl.when(i == 0)
        def _init():
            s1_ref[...] = s1i_ref[...]
            s2_ref[...] = s2i_ref[...]

        s1_ref[...] += jnp.sum(y, axis=0, keepdims=True)
        s2_ref[...] += jnp.sum(y * y, axis=0, keepdims=True)

    in_specs = [
        pl.BlockSpec((_EB, c), lambda i: (i + off, 0)),
        pl.BlockSpec((_NTAP, _EB, c), lambda i: (0, i, 0)),
        pl.BlockSpec((5 * c, c), lambda i: (0, 0)),
        pl.BlockSpec((1, c), lambda i: (0, 0)),
        pl.BlockSpec((1, c), lambda i: (0, 0)),
    ]
    args = [xt, taps_h, wc, s1_init, s2_init]
    aliases = {}
    if y_prev is not None:
        in_specs.append(pl.BlockSpec(memory_space=pl.ANY))
        args.append(y_prev)
        aliases = {5: 0}

    return pl.pallas_call(
        body,
        grid=(nbh,),
        in_specs=in_specs,
        out_specs=[
            pl.BlockSpec((_EB, c), lambda i: (i + off, 0)),
            pl.BlockSpec((1, c), lambda i: (0, 0)),
            pl.BlockSpec((1, c), lambda i: (0, 0)),
        ],
        out_shape=[
            jax.ShapeDtypeStruct((e, c), jnp.float32),
            jax.ShapeDtypeStruct((1, c), jnp.float32),
            jax.ShapeDtypeStruct((1, c), jnp.float32),
        ],
        input_output_aliases=aliases,
    )(*args)


def _tc_bn_relu(y, scale, shift):
    e, c = y.shape

    def body(y_ref, sc_ref, sh_ref, o_ref):
        z = jnp.maximum(y_ref[...] * sc_ref[...] + sh_ref[...], 0.0)
        o_ref[...] = z.T

    return pl.pallas_call(
        body,
        grid=(e // _EB2,),
        in_specs=[
            pl.BlockSpec((_EB2, c), lambda i: (i, 0)),
            pl.BlockSpec((1, c), lambda i: (0, 0)),
            pl.BlockSpec((1, c), lambda i: (0, 0)),
        ],
        out_specs=pl.BlockSpec((c, _EB2), lambda i: (0, i)),
        out_shape=jax.ShapeDtypeStruct((c, e), jnp.float32),
    )(y, scale, shift)


def kernel(x, gemm, W, b, gamma, beta):
    _, c_in, e = x.shape
    c_out = W.shape[0]

    eh = e // 2
    xt = jnp.swapaxes(x[0], 0, 1)
    idx0 = jnp.swapaxes(gemm[0, :eh], 0, 1).reshape(-1)
    idx1 = jnp.swapaxes(gemm[0, eh:], 0, 1).reshape(-1)
    taps0 = _sc_gather(xt, idx0).reshape(_NTAP, eh, c_in)
    taps1 = _sc_gather(xt, idx1).reshape(_NTAP, eh, c_in)

    wc = jnp.transpose(W, (2, 1, 0)).reshape(5 * c_in, c_out).astype(jnp.bfloat16)
    zc = jnp.zeros((1, c_out), jnp.float32)
    y0, s1a, s2a = _tc_conv_stats(xt, taps0, wc, None, zc, zc, 0, 2)
    y, s1, s2 = _tc_conv_stats(xt, taps1, wc, y0, s1a, s2a, 1, 2)

    mean = s1[0] / e
    var = s2[0] / e - mean * mean
    inv = gamma / jnp.sqrt(var + 1e-5)
    scale = inv[None]
    shift = (beta - mean * inv)[None]

    out = _tc_bn_relu(y, scale, shift)[None]
    return (out, gemm)


# speedup vs baseline: 5.2435x; 1.0321x over previous
import functools

import jax
import jax.numpy as jnp
from jax import lax
from jax.experimental import pallas as pl
from jax.experimental.pallas import tpu as pltpu
from jax.experimental.pallas import tpu_sc as plsc

_NTAP = 4
_NW = 32
_KC = 128
_EB = 2000
_EB2 = 3200


def _sc_gather(table, idx):
    n, = idx.shape
    _, c = table.shape
    per_w = n // _NW
    nfull = per_w // _KC
    tail = per_w - nfull * _KC
    ns = 6
    nf = 4
    assert nfull % ns == 0 and nfull >= 2 * ns

    mesh = plsc.VectorSubcoreMesh(core_axis_name="c", subcore_axis_name="s")

    @functools.partial(
        pl.kernel,
        mesh=mesh,
        out_type=jax.ShapeDtypeStruct((n, c), table.dtype),
        scratch_types=[
            pltpu.VMEM((per_w,), jnp.int32),
        ] + [pltpu.VMEM((_KC, c), table.dtype)] * ns
          + [pltpu.SemaphoreType.DMA] * (2 * ns),
    )
    def gather_kernel(table_hbm, idx_hbm, out_hbm, idx_v, *bufs):
        rows = bufs[:ns]
        semg = bufs[ns:2 * ns]
        semw = bufs[2 * ns:3 * ns]
        wid = lax.axis_index("s") * 2 + lax.axis_index("c")
        base_w = wid * per_w

        pltpu.sync_copy(idx_hbm.at[pl.ds(base_w, per_w)], idx_v)

        def g_idx(m):
            return idx_v.at[pl.ds(m * _KC, _KC)]

        for m in range(nf):
            pltpu.async_copy(table_hbm.at[g_idx(m)], rows[m], semg[m])

        def step(j, carry):
            for k in range(ns):
                m = ns * j + k
                sl = (k + nf) % ns

                @pl.when(m + nf < nfull)
                def _launch():
                    @pl.when(m + nf >= ns)
                    def _reclaim():
                        pltpu.make_async_copy(
                            rows[sl], out_hbm.at[pl.ds(base_w, _KC)],
                            semw[sl]).wait()
                    pltpu.async_copy(
                        table_hbm.at[g_idx(m + nf)], rows[sl], semg[sl])

                pltpu.make_async_copy(
                    table_hbm.at[g_idx(m)], rows[k], semg[k]).wait()
                pltpu.async_copy(
                    rows[k], out_hbm.at[pl.ds(base_w + m * _KC, _KC)], semw[k])
            return carry

        lax.fori_loop(0, nfull // ns, step, 0)
        for m in range(nfull - ns, nfull):
            pltpu.make_async_copy(
                rows[m % ns], out_hbm.at[pl.ds(base_w, _KC)],
                semw[m % ns]).wait()
        if tail:
            tb = base_w + nfull * _KC
            pltpu.async_copy(
                table_hbm.at[idx_v.at[pl.ds(nfull * _KC, tail)]],
                rows[0].at[pl.ds(0, tail)], semg[0]).wait()
            pltpu.sync_copy(
                rows[0].at[pl.ds(0, tail)], out_hbm.at[pl.ds(tb, tail)])

    return gather_kernel(table, idx)


def _tc_conv_stats(xt, taps_h, wc, y_prev, s1_init, s2_init, half, nhalf):
    e, c = xt.shape
    nbh = (e // nhalf) // _EB
    off = half * nbh

    def body(xt_ref, taps_ref, wc_ref, s1i_ref, s2i_ref, *rest):
        y_ref, s1_ref, s2_ref = rest[-3:]
        i = pl.program_id(0)
        bf = jnp.bfloat16
        x = xt_ref[...].astype(bf)
        a = taps_ref[0]
        bb = taps_ref[1]
        cc = taps_ref[2]
        dd = taps_ref[3]
        h = jnp.concatenate(
            [x, (a + cc).astype(bf), (bb + dd).astype(bf),
             jnp.abs(a - cc).astype(bf), jnp.abs(bb - dd).astype(bf)], axis=1)
        y = jnp.dot(h, wc_ref[...], preferred_element_type=jnp.float32)
        y_ref[...] = y.astype(jnp.bfloat16)

        @pl.when(i == 0)
        def _init():
            s1_ref[...] = s1i_ref[...]
            s2_ref[...] = s2i_ref[...]

        s1_ref[...] += jnp.sum(y, axis=0, keepdims=True)
        s2_ref[...] += jnp.sum(y * y, axis=0, keepdims=True)

    in_specs = [
        pl.BlockSpec((_EB, c), lambda i: (i + off, 0)),
        pl.BlockSpec((_NTAP, _EB, c), lambda i: (0, i, 0)),
        pl.BlockSpec((5 * c, c), lambda i: (0, 0)),
        pl.BlockSpec((1, c), lambda i: (0, 0)),
        pl.BlockSpec((1, c), lambda i: (0, 0)),
    ]
    args = [xt, taps_h, wc, s1_init, s2_init]
    aliases = {}
    if y_prev is not None:
        in_specs.append(pl.BlockSpec(memory_space=pl.ANY))
        args.append(y_prev)
        aliases = {5: 0}

    return pl.pallas_call(
        body,
        grid=(nbh,),
        in_specs=in_specs,
        out_specs=[
            pl.BlockSpec((_EB, c), lambda i: (i + off, 0)),
            pl.BlockSpec((1, c), lambda i: (0, 0)),
            pl.BlockSpec((1, c), lambda i: (0, 0)),
        ],
        out_shape=[
            jax.ShapeDtypeStruct((e, c), jnp.bfloat16),
            jax.ShapeDtypeStruct((1, c), jnp.float32),
            jax.ShapeDtypeStruct((1, c), jnp.float32),
        ],
        input_output_aliases=aliases,
    )(*args)


def _tc_bn_relu(y, scale, shift):
    e, c = y.shape

    def body(y_ref, sc_ref, sh_ref, o_ref):
        yv = y_ref[...].astype(jnp.float32)
        z = jnp.maximum(yv * sc_ref[...] + sh_ref[...], 0.0)
        o_ref[...] = z.T

    return pl.pallas_call(
        body,
        grid=(e // _EB2,),
        in_specs=[
            pl.BlockSpec((_EB2, c), lambda i: (i, 0)),
            pl.BlockSpec((1, c), lambda i: (0, 0)),
            pl.BlockSpec((1, c), lambda i: (0, 0)),
        ],
        out_specs=pl.BlockSpec((c, _EB2), lambda i: (0, i)),
        out_shape=jax.ShapeDtypeStruct((c, e), jnp.float32),
    )(y, scale, shift)


def kernel(x, gemm, W, b, gamma, beta):
    _, c_in, e = x.shape
    c_out = W.shape[0]

    eh = e // 2
    xt = jnp.swapaxes(x[0], 0, 1)
    idx0 = jnp.swapaxes(gemm[0, :eh], 0, 1).reshape(-1)
    idx1 = jnp.swapaxes(gemm[0, eh:], 0, 1).reshape(-1)
    taps0 = _sc_gather(xt, idx0).reshape(_NTAP, eh, c_in)
    taps1 = _sc_gather(xt, idx1).reshape(_NTAP, eh, c_in)

    wc = jnp.transpose(W, (2, 1, 0)).reshape(5 * c_in, c_out).astype(jnp.bfloat16)
    zc = jnp.zeros((1, c_out), jnp.float32)
    y0, s1a, s2a = _tc_conv_stats(xt, taps0, wc, None, zc, zc, 0, 2)
    y, s1, s2 = _tc_conv_stats(xt, taps1, wc, y0, s1a, s2a, 1, 2)

    mean = s1[0] / e
    var = s2[0] / e - mean * mean
    inv = gamma / jnp.sqrt(var + 1e-5)
    scale = inv[None]
    shift = (beta - mean * inv)[None]

    out = _tc_bn_relu(y, scale, shift)[None]
    return (out, gemm)
